# speedup vs baseline: 39.7245x; 39.7245x over previous
import jax
import jax.numpy as jnp
from jax.experimental import pallas as pl
from jax.experimental.pallas import tpu as pltpu

_S = 512
_B = 2000


def _pass1(x_ref, seg_ref, wt_ref, b_ref, scores_ref, m_ref, denom_ref):
    i = pl.program_id(0)

    @pl.when(i == 0)
    def _init():
        m_ref[...] = jnp.full_like(m_ref, -jnp.inf)
        denom_ref[...] = jnp.zeros_like(denom_ref)

    x = x_ref[...]
    s = jnp.dot(x, wt_ref[...], preferred_element_type=jnp.float32) + b_ref[...]
    scores_ref[...] = s

    m_old = m_ref[0:1, :]
    m_new = jnp.maximum(m_old, jnp.max(s, axis=0, keepdims=True))
    scale = jnp.where(m_new == m_old, 1.0, jnp.exp(m_old - m_new))
    e = jnp.exp(s - m_new)

    seg = seg_ref[0, 0, :]
    ot = (jax.lax.broadcasted_iota(jnp.int32, (_S, s.shape[0]), 0)
          == seg[None, :]).astype(jnp.float32)
    dblk = jnp.dot(ot, e, preferred_element_type=jnp.float32)
    denom_ref[...] = denom_ref[...] * scale + dblk
    m_ref[...] = jnp.broadcast_to(m_new, m_ref.shape)


def _pass2(x_ref, sc_ref, seg_ref, m_ref, d_ref, attn_ref, pooled_ref):
    i = pl.program_id(0)

    @pl.when(i == 0)
    def _init():
        pooled_ref[...] = jnp.zeros_like(pooled_ref)

    s = sc_ref[...]
    e = jnp.exp(s - m_ref[0:1, :])
    seg = seg_ref[0, 0, :]
    ot = (jax.lax.broadcasted_iota(jnp.int32, (_S, s.shape[0]), 0)
          == seg[None, :]).astype(jnp.float32)
    dg = jax.lax.dot_general(ot, d_ref[...], (((0,), (0,)), ((), ())),
                             preferred_element_type=jnp.float32)
    attn = e / jnp.maximum(dg, 1e-16)
    attn_ref[...] = attn
    c = jnp.mean(attn, axis=1, keepdims=True)
    pooled_ref[...] += jnp.dot(ot, x_ref[...] * c,
                               preferred_element_type=jnp.float32)


def kernel(x, batch_indices, W, b, temperature):
    n, d = x.shape
    h = W.shape[0]
    nblk = n // _B
    assert nblk * _B == n

    wt = (W / temperature).T.astype(jnp.float32)
    b2 = (b / temperature).reshape(1, h).astype(jnp.float32)
    seg3 = batch_indices.astype(jnp.int32).reshape(nblk, 1, _B)

    params = pltpu.CompilerParams(dimension_semantics=("arbitrary",))

    scores, m, denom = pl.pallas_call(
        _pass1,
        grid=(nblk,),
        in_specs=[
            pl.BlockSpec((_B, d), lambda i: (i, 0)),
            pl.BlockSpec((1, 1, _B), lambda i: (i, 0, 0)),
            pl.BlockSpec((d, h), lambda i: (0, 0)),
            pl.BlockSpec((1, h), lambda i: (0, 0)),
        ],
        out_specs=[
            pl.BlockSpec((_B, h), lambda i: (i, 0)),
            pl.BlockSpec((8, h), lambda i: (0, 0)),
            pl.BlockSpec((_S, h), lambda i: (0, 0)),
        ],
        out_shape=[
            jax.ShapeDtypeStruct((n, h), jnp.float32),
            jax.ShapeDtypeStruct((8, h), jnp.float32),
            jax.ShapeDtypeStruct((_S, h), jnp.float32),
        ],
        compiler_params=params,
    )(x, seg3, wt, b2)

    attn, pooled = pl.pallas_call(
        _pass2,
        grid=(nblk,),
        in_specs=[
            pl.BlockSpec((_B, d), lambda i: (i, 0)),
            pl.BlockSpec((_B, h), lambda i: (i, 0)),
            pl.BlockSpec((1, 1, _B), lambda i: (i, 0, 0)),
            pl.BlockSpec((8, h), lambda i: (0, 0)),
            pl.BlockSpec((_S, h), lambda i: (0, 0)),
        ],
        out_specs=[
            pl.BlockSpec((_B, h), lambda i: (i, 0)),
            pl.BlockSpec((_S, d), lambda i: (0, 0)),
        ],
        out_shape=[
            jax.ShapeDtypeStruct((n, h), jnp.float32),
            jax.ShapeDtypeStruct((_S, d), jnp.float32),
        ],
        compiler_params=params,
    )(x, scores, seg3, m, denom)

    return (pooled, attn.T)
